# baseline (speedup 1.0000x reference)
import jax
import jax.numpy as jnp
from jax.experimental import pallas as pl
from jax.experimental.pallas import tpu as pltpu

NUM_PATCHES = 576
LATENT_DIM = 768
BATCH = 64

BB = 4
NB = 4
NSTEP = BATCH // BB


def _pipeline(x_hbm, pos_ref, out_hbm, xbuf, obuf, in_sem, out_sem):
    def in_copy(k, s):
        return pltpu.make_async_copy(
            x_hbm.at[pl.ds(k * BB, BB)], xbuf.at[s], in_sem.at[s])

    def out_copy(k, s):
        return pltpu.make_async_copy(
            obuf.at[s], out_hbm.at[pl.ds(k * BB, BB)], out_sem.at[s])

    for s in range(NB):
        in_copy(s, s).start()

    for k in range(NSTEP):
        s = k % NB
        in_copy(k, s).wait()
        if k >= NB:
            out_copy(k - NB, s).wait()
        obuf[s] = xbuf[s] + pos_ref[...]
        out_copy(k, s).start()
        if k + NB < NSTEP:
            in_copy(k + NB, s).start()

    for k in range(NSTEP - NB, NSTEP):
        out_copy(k, k % NB).wait()


def kernel(x, pos_table):
    return pl.pallas_call(
        _pipeline,
        in_specs=[
            pl.BlockSpec(memory_space=pltpu.HBM),
            pl.BlockSpec(memory_space=pltpu.VMEM),
        ],
        out_specs=pl.BlockSpec(memory_space=pltpu.HBM),
        out_shape=jax.ShapeDtypeStruct((BATCH, NUM_PATCHES, LATENT_DIM), x.dtype),
        scratch_shapes=[
            pltpu.VMEM((NB, BB, NUM_PATCHES, LATENT_DIM), jnp.float32),
            pltpu.VMEM((NB, BB, NUM_PATCHES, LATENT_DIM), jnp.float32),
            pltpu.SemaphoreType.DMA((NB,)),
            pltpu.SemaphoreType.DMA((NB,)),
        ],
    )(x, pos_table)
